# Initial kernel scaffold; baseline (speedup 1.0000x reference)
import jax
import jax.numpy as jnp
from jax.experimental import pallas as pl


def kernel(encoded_x, codebooks):
    raise NotImplementedError("write your pallas kernel here")


# speedup vs baseline: 1.3392x; 1.3392x over previous
import functools

import jax
import jax.numpy as jnp
from jax import lax
from jax.experimental import pallas as pl
from jax.experimental.pallas import tpu as pltpu
from jax.experimental.pallas import tpu_sc as plsc

N_Q = 4
K = 8192
D = 64
N = 8192

TILE = 256
NT = N // TILE

SC_NC = 2
SC_NS = 16
SC_L = 16
SC_NW = SC_NC * SC_NS
SC_BPW = N // SC_NW


def _stage_body(r_ref, rn_ref, cb_ref, cbn_ref, idx_ref):
    rt = r_ref[...]
    m2 = lax.dot_general(rt, cb_ref[...],
                         (((1,), (0,)), ((), ())),
                         preferred_element_type=jnp.float32)
    d2 = (rn_ref[...] - m2) + cbn_ref[...]
    idx_ref[...] = jnp.argmin(d2, axis=1, keepdims=True).astype(jnp.int32)


def _stage_tc(r, rn, cb2, cbn):
    return pl.pallas_call(
        _stage_body,
        grid=(NT,),
        in_specs=[
            pl.BlockSpec((TILE, D), lambda i: (i, 0)),
            pl.BlockSpec((TILE, 1), lambda i: (i, 0)),
            pl.BlockSpec((D, K), lambda i: (0, 0)),
            pl.BlockSpec((1, K), lambda i: (0, 0)),
        ],
        out_specs=pl.BlockSpec((TILE, 1), lambda i: (i, 0)),
        out_shape=jax.ShapeDtypeStruct((N, 1), jnp.int32),
    )(r, rn, cb2, cbn)


def _gather_update_sc(cb, idx, r):
    mesh = plsc.VectorSubcoreMesh(core_axis_name="c", subcore_axis_name="s")

    @functools.partial(
        pl.kernel,
        mesh=mesh,
        out_type=[
            jax.ShapeDtypeStruct((N, 128), jnp.float32),
            jax.ShapeDtypeStruct((N, D), jnp.float32),
        ],
        scratch_types=[
            pltpu.VMEM((SC_BPW,), jnp.int32),
            pltpu.VMEM((SC_BPW, 128), jnp.float32),
            pltpu.VMEM((SC_BPW, D), jnp.float32),
            pltpu.SemaphoreType.DMA,
        ],
    )
    def k(cb_hbm, idx_hbm, r_hbm, quant_out, rnew_out, idx_v, rows_v, r_v, sem):
        wid = lax.axis_index("s") * SC_NC + lax.axis_index("c")
        base = wid * SC_BPW
        pltpu.sync_copy(idx_hbm.at[pl.ds(base, SC_BPW)], idx_v)
        pltpu.sync_copy(r_hbm.at[pl.ds(base, SC_BPW)], r_v)
        pltpu.async_copy(cb_hbm.at[idx_v], rows_v, sem).wait()
        pltpu.sync_copy(rows_v, quant_out.at[pl.ds(base, SC_BPW)])

        def body(i, carry):
            for c in range(D // SC_L):
                q = rows_v[i, pl.ds(c * SC_L, SC_L)]
                rv = r_v[i, pl.ds(c * SC_L, SC_L)]
                t = q - rv
                qst = rv + t
                r_v[i, pl.ds(c * SC_L, SC_L)] = rv - qst
            return carry

        lax.fori_loop(0, SC_BPW, body, 0)
        pltpu.sync_copy(r_v, rnew_out.at[pl.ds(base, SC_BPW)])

    return k(cb, idx, r)


def _epilogue_body(x_ref, q0_ref, q1_ref, q2_ref, q3_ref, out_ref, loss_ref):
    r = x_ref[...]
    acc = jnp.zeros_like(r)
    sums = []
    for q_ref in (q0_ref, q1_ref, q2_ref, q3_ref):
        qv = q_ref[:, :D]
        t = qv - r
        qst = r + t
        acc = acc + qst
        r = r - qst
        sums.append(jnp.sum(t * t))
    out_ref[...] = acc
    row = lax.broadcasted_iota(jnp.int32, (8, 128), 0)
    lmat = (jnp.where(row == 0, sums[0], 0.0)
            + jnp.where(row == 1, sums[1], 0.0)
            + jnp.where(row == 2, sums[2], 0.0)
            + jnp.where(row == 3, sums[3], 0.0))

    @pl.when(pl.program_id(0) == 0)
    def _():
        loss_ref[...] = lmat

    @pl.when(pl.program_id(0) != 0)
    def _():
        loss_ref[...] += lmat


def _epilogue(x, q0, q1, q2, q3):
    tok_spec = pl.BlockSpec((TILE, D), lambda i: (i, 0))
    quant_spec = pl.BlockSpec((TILE, 128), lambda i: (i, 0))
    return pl.pallas_call(
        _epilogue_body,
        grid=(NT,),
        in_specs=[tok_spec] + [quant_spec] * 4,
        out_specs=[
            pl.BlockSpec((TILE, D), lambda i: (i, 0)),
            pl.BlockSpec((8, 128), lambda i: (0, 0)),
        ],
        out_shape=[
            jax.ShapeDtypeStruct((N, D), jnp.float32),
            jax.ShapeDtypeStruct((8, 128), jnp.float32),
        ],
    )(x, q0, q1, q2, q3)


def kernel(encoded_x, codebooks):
    x = jnp.transpose(encoded_x, (0, 3, 2, 1)).reshape(-1, encoded_x.shape[1])
    cbn = jnp.sum(codebooks * codebooks, axis=-1)
    cbp = jnp.pad(codebooks, ((0, 0), (0, 0), (0, 128 - D)))
    r = x
    idxs = []
    quants = []
    for q in range(N_Q):
        rn = jnp.sum(r * r, axis=1, keepdims=True)
        idx2 = _stage_tc(r, rn, (2.0 * codebooks[q]).T, cbn[q].reshape(1, K))
        idx = idx2.reshape(N)
        quant, r = _gather_update_sc(cbp[q], idx, r)
        idxs.append(idx)
        quants.append(quant)
    qout, loss_mat = _epilogue(x, *quants)
    indices = jnp.stack(idxs, axis=-1)
    losses = loss_mat[:N_Q, 0] / jnp.float32(N * D)
    return qout, indices, losses
